# Initial kernel scaffold; baseline (speedup 1.0000x reference)
import jax
import jax.numpy as jnp
from jax.experimental import pallas as pl


def kernel(x, edge_index, edge_index2, W1_self, W1_neigh, b1, W2_self, W2_neigh, b2, num_bits, num_grad_bits):
    raise NotImplementedError("write your pallas kernel here")


# speedup vs baseline: 5.8824x; 5.8824x over previous
import functools

import jax
import jax.numpy as jnp
from jax import lax
from jax.experimental import pallas as pl
from jax.experimental.pallas import tpu as pltpu
from jax.experimental.pallas import tpu_sc as plsc

N = 10000
E = 160000
D_IN = 256
D_H = 512
D_OUT = 128

NC = 2
NS = 16
CH = 128
EP = 163840
NPAD = 10240
ZR = NPAD // NS

_mesh = plsc.VectorSubcoreMesh(core_axis_name="c", subcore_axis_name="s")


@functools.partial(
    pl.kernel,
    mesh=_mesh,
    out_type=(
        jax.ShapeDtypeStruct((NPAD, 128), jnp.float32),
        jax.ShapeDtypeStruct((NPAD, 128), jnp.float32),
    ),
    scratch_types=[
        pltpu.VMEM((CH,), jnp.int32),
        pltpu.VMEM((CH, 128), jnp.float32),
        pltpu.VMEM_SHARED((NPAD, 128), jnp.float32),
    ],
)
def _sc_counts(dst1p, dst2p, zerosF, onesF, cnt1, cnt2,
               dst_v, buf_v, acc_sh):
    c = lax.axis_index("c")
    s = lax.axis_index("s")
    pltpu.sync_copy(zerosF, buf_v)
    for k in range(ZR // CH):
        pltpu.sync_copy(buf_v, acc_sh.at[pl.ds(s * ZR + k * CH, CH)])
    pltpu.sync_copy(onesF, buf_v)
    plsc.subcore_barrier()

    per_tile = EP // NS
    base = s * per_tile

    def chunk1(i, carry):
        pltpu.sync_copy(dst1p.at[pl.ds(base + i * CH, CH)], dst_v)
        pltpu.sync_copy(buf_v, acc_sh.at[dst_v], add=True)
        return carry

    def chunk2(i, carry):
        pltpu.sync_copy(dst2p.at[pl.ds(base + i * CH, CH)], dst_v)
        pltpu.sync_copy(buf_v, acc_sh.at[dst_v], add=True)
        return carry

    @pl.when(c == 0)
    def _():
        lax.fori_loop(0, per_tile // CH, chunk1, 0)

    @pl.when(c == 1)
    def _():
        lax.fori_loop(0, per_tile // CH, chunk2, 0)

    plsc.subcore_barrier()
    for k in range(ZR // CH):
        sl = pl.ds(s * ZR + k * CH, CH)
        pltpu.sync_copy(acc_sh.at[sl], buf_v)

        @pl.when(c == 0)
        def _():
            pltpu.sync_copy(buf_v, cnt1.at[sl])

        @pl.when(c == 1)
        def _():
            pltpu.sync_copy(buf_v, cnt2.at[sl])


@functools.partial(
    pl.kernel,
    mesh=_mesh,
    out_type=(
        jax.ShapeDtypeStruct((NPAD, 128), jnp.float32),
        jax.ShapeDtypeStruct((NPAD, 128), jnp.float32),
    ),
    scratch_types=[
        pltpu.VMEM((CH,), jnp.int32),
        pltpu.VMEM((CH,), jnp.int32),
        pltpu.VMEM((CH, 128), jnp.float32),
        pltpu.VMEM_SHARED((NPAD, 128), jnp.float32),
        pltpu.SemaphoreType.DMA,
    ],
)
def _sc_segsum_feat_split(xflat, srcp, dstp, zerosF, aggL, aggR,
                          idx_v, dst_v, rows_v, acc_sh, sem):
    c = lax.axis_index("c")
    s = lax.axis_index("s")
    pltpu.sync_copy(zerosF, rows_v)
    for k in range(ZR // CH):
        pltpu.sync_copy(rows_v, acc_sh.at[pl.ds(s * ZR + k * CH, CH)])
    plsc.subcore_barrier()

    per_tile = EP // NS
    base = s * per_tile
    coff = c * N

    def chunk(i, carry):
        off = base + i * CH
        pltpu.sync_copy(srcp.at[pl.ds(off, CH)], idx_v)
        pltpu.sync_copy(dstp.at[pl.ds(off, CH)], dst_v)
        for j in range(CH // 16):
            idx_v[pl.ds(j * 16, 16)] = idx_v[pl.ds(j * 16, 16)] + coff
        pltpu.async_copy(xflat.at[idx_v], rows_v, sem).wait()
        pltpu.sync_copy(rows_v, acc_sh.at[dst_v], add=True)
        return carry

    lax.fori_loop(0, per_tile // CH, chunk, 0)
    plsc.subcore_barrier()

    for k in range(ZR // CH):
        sl = pl.ds(s * ZR + k * CH, CH)
        pltpu.sync_copy(acc_sh.at[sl], rows_v)

        @pl.when(c == 0)
        def _():
            pltpu.sync_copy(rows_v, aggL.at[sl])

        @pl.when(c == 1)
        def _():
            pltpu.sync_copy(rows_v, aggR.at[sl])


@functools.partial(
    pl.kernel,
    mesh=_mesh,
    out_type=(
        jax.ShapeDtypeStruct((NPAD, 128), jnp.float32),
        jax.ShapeDtypeStruct((NPAD, 128), jnp.float32),
    ),
    scratch_types=[
        pltpu.VMEM((CH,), jnp.int32),
        pltpu.VMEM((CH,), jnp.int32),
        pltpu.VMEM((CH, 128), jnp.float32),
        pltpu.VMEM_SHARED((NPAD, 128), jnp.float32),
        pltpu.SemaphoreType.DMA,
    ],
)
def _sc_segsum_edge_split(table, srcp, dstp, zerosF, p0, p1,
                          idx_v, dst_v, rows_v, acc_sh, sem):
    c = lax.axis_index("c")
    s = lax.axis_index("s")
    pltpu.sync_copy(zerosF, rows_v)
    for k in range(ZR // CH):
        pltpu.sync_copy(rows_v, acc_sh.at[pl.ds(s * ZR + k * CH, CH)])
    plsc.subcore_barrier()

    per_tile = EP // (NC * NS)
    base = (c * NS + s) * per_tile

    def chunk(i, carry):
        off = base + i * CH
        pltpu.sync_copy(srcp.at[pl.ds(off, CH)], idx_v)
        pltpu.sync_copy(dstp.at[pl.ds(off, CH)], dst_v)
        pltpu.async_copy(table.at[idx_v], rows_v, sem).wait()
        pltpu.sync_copy(rows_v, acc_sh.at[dst_v], add=True)
        return carry

    lax.fori_loop(0, per_tile // CH, chunk, 0)
    plsc.subcore_barrier()

    for k in range(ZR // CH):
        sl = pl.ds(s * ZR + k * CH, CH)
        pltpu.sync_copy(acc_sh.at[sl], rows_v)

        @pl.when(c == 0)
        def _():
            pltpu.sync_copy(rows_v, p0.at[sl])

        @pl.when(c == 1)
        def _():
            pltpu.sync_copy(rows_v, p1.at[sl])


BM = 1000


def _tc_layer1_body(x_ref, aggL_ref, aggR_ref, cnt_ref,
                    w1s_ref, w1nL_ref, w1nR_ref, b1_ref, w2s_ref, w2n_ref,
                    s_ref, t_ref, sums_ref, acc_ref):
    i = pl.program_id(0)
    c1 = jnp.maximum(cnt_ref[:, 0:1], 1.0)
    aL = aggL_ref[...] / c1
    aR = aggR_ref[...] / c1
    h = jnp.dot(x_ref[...], w1s_ref[...], preferred_element_type=jnp.float32)
    h = h + jnp.dot(aL, w1nL_ref[...], preferred_element_type=jnp.float32)
    h = h + jnp.dot(aR, w1nR_ref[...], preferred_element_type=jnp.float32)
    h = h + b1_ref[...]
    h = jnp.maximum(h, 0.0)
    s_ref[...] = jnp.dot(h, w2s_ref[...], preferred_element_type=jnp.float32)
    t_ref[...] = jnp.dot(h, w2n_ref[...], preferred_element_type=jnp.float32)
    bsum = jnp.sum(h)
    bsq = jnp.sum(h * h)

    @pl.when(i == 0)
    def _():
        acc_ref[0] = bsum
        acc_ref[1] = bsq

    @pl.when(i > 0)
    def _():
        acc_ref[0] += bsum
        acc_ref[1] += bsq

    @pl.when(i == pl.num_programs(0) - 1)
    def _():
        sums_ref[0, 0] = acc_ref[0]
        sums_ref[0, 1] = acc_ref[1]


def _tc_layer2_body(s_ref, p0_ref, p1_ref, cnt_ref,
                    sums_ref, w2s_ref, w2n_ref, b2_ref, out_ref):
    tot = float(N * D_H)
    ssum = sums_ref[0, 0]
    ssq = sums_ref[0, 1]
    mu = ssum / tot
    var = ssq / tot - mu * mu
    inv = 1.0 / jnp.sqrt(var + 1e-5)
    cs_self = jnp.sum(w2s_ref[...], axis=0, keepdims=True)
    cs_neigh = jnp.sum(w2n_ref[...], axis=0, keepdims=True)
    seg = p0_ref[...] + p1_ref[...]
    c2 = cnt_ref[:, 0:1]
    denom = jnp.maximum(c2, 1.0)
    aggterm = (seg - (c2 * mu) * cs_neigh) / denom
    out_ref[...] = (s_ref[...] - mu * cs_self + aggterm) * inv + b2_ref[...]


def kernel(x, edge_index, edge_index2, W1_self, W1_neigh, b1,
           W2_self, W2_neigh, b2, num_bits, num_grad_bits):
    f32 = jnp.float32
    xflat = jnp.concatenate([x[:, :128], x[:, 128:]], axis=0)
    pad = EP - E
    pidx = jnp.arange(pad, dtype=jnp.int32)
    psrc = pidx % N
    pdst = N + (pidx % 16)
    src1p = jnp.concatenate([edge_index[0], psrc])
    dst1p = jnp.concatenate([edge_index[1], pdst])
    src2p = jnp.concatenate([edge_index2[0], psrc])
    dst2p = jnp.concatenate([edge_index2[1], pdst])
    zerosF = jnp.zeros((CH, 128), f32)
    onesF = jnp.ones((CH, 128), f32)

    cnt1, cnt2 = _sc_counts(dst1p, dst2p, zerosF, onesF)
    aggL, aggR = _sc_segsum_feat_split(xflat, src1p, dst1p, zerosF)

    grid = N // BM
    full = lambda shape: pl.BlockSpec(shape, lambda i: (0, 0))
    row_blk = lambda w: pl.BlockSpec((BM, w), lambda i: (i, 0))
    S, T, sums = pl.pallas_call(
        _tc_layer1_body,
        grid=(grid,),
        in_specs=[
            row_blk(D_IN), row_blk(128), row_blk(128), row_blk(128),
            full((D_IN, D_H)), full((128, D_H)), full((128, D_H)),
            full((1, D_H)), full((D_H, D_OUT)), full((D_H, D_OUT)),
        ],
        out_specs=[
            row_blk(D_OUT), row_blk(D_OUT),
            pl.BlockSpec(memory_space=pltpu.SMEM),
        ],
        out_shape=[
            jax.ShapeDtypeStruct((N, D_OUT), f32),
            jax.ShapeDtypeStruct((N, D_OUT), f32),
            jax.ShapeDtypeStruct((1, 2), f32),
        ],
        scratch_shapes=[pltpu.SMEM((2,), f32)],
    )(x, aggL, aggR, cnt1,
      W1_self, W1_neigh[:128], W1_neigh[128:], b1.reshape(1, D_H),
      W2_self, W2_neigh)

    p0, p1 = _sc_segsum_edge_split(T, src2p, dst2p, zerosF)

    out = pl.pallas_call(
        _tc_layer2_body,
        grid=(grid,),
        in_specs=[
            row_blk(D_OUT), row_blk(D_OUT), row_blk(D_OUT), row_blk(128),
            pl.BlockSpec(memory_space=pltpu.SMEM),
            full((D_H, D_OUT)), full((D_H, D_OUT)), full((1, D_OUT)),
        ],
        out_specs=row_blk(D_OUT),
        out_shape=jax.ShapeDtypeStruct((N, D_OUT), f32),
    )(S, p0, p1, cnt2, sums, W2_self, W2_neigh, b2.reshape(1, D_OUT))

    return out


# speedup vs baseline: 10.2289x; 1.7389x over previous
import functools

import jax
import jax.numpy as jnp
from jax import lax
from jax.experimental import pallas as pl
from jax.experimental.pallas import tpu as pltpu
from jax.experimental.pallas import tpu_sc as plsc

N = 10000
E = 160000
D_IN = 256
D_H = 512
D_OUT = 128

NC = 2
NS = 16
CH = 128
EP = 163840
ECH = EP // CH
NPAD = 10240
ZR = NPAD // NS

_mesh = plsc.VectorSubcoreMesh(core_axis_name="c", subcore_axis_name="s")


@functools.partial(
    pl.kernel,
    mesh=_mesh,
    out_type=(
        jax.ShapeDtypeStruct((NPAD, 128), jnp.float32),
        jax.ShapeDtypeStruct((NPAD, 128), jnp.float32),
    ),
    scratch_types=[
        pltpu.VMEM((ECH // NS, CH), jnp.int32),
        pltpu.VMEM((CH, 128), jnp.float32),
        pltpu.VMEM_SHARED((NPAD, 128), jnp.float32),
        pltpu.SemaphoreType.DMA,
    ],
)
def _sc_counts(dst1p, dst2p, zerosF, onesF, cnt1, cnt2,
               dst2d, buf_v, acc_sh, sem_s):
    c = lax.axis_index("c")
    s = lax.axis_index("s")
    nch = ECH // NS
    pltpu.sync_copy(zerosF, buf_v)
    for k in range(ZR // CH):
        pltpu.sync_copy(buf_v, acc_sh.at[pl.ds(s * ZR + k * CH, CH)])

    @pl.when(c == 0)
    def _():
        pltpu.sync_copy(dst1p.at[pl.ds(s * nch, nch)], dst2d)

    @pl.when(c == 1)
    def _():
        pltpu.sync_copy(dst2p.at[pl.ds(s * nch, nch)], dst2d)

    pltpu.sync_copy(onesF, buf_v)
    plsc.subcore_barrier()

    AHEAD = 4

    def chunk(i, carry):
        pltpu.async_copy(buf_v, acc_sh.at[dst2d.at[i]], sem_s, add=True)

        @pl.when(i >= AHEAD)
        def _():
            pltpu.make_async_copy(buf_v, acc_sh.at[dst2d.at[i]], sem_s).wait()

        return carry

    lax.fori_loop(0, nch, chunk, 0)
    for _ in range(AHEAD):
        pltpu.make_async_copy(buf_v, acc_sh.at[dst2d.at[0]], sem_s).wait()

    plsc.subcore_barrier()
    for k in range(ZR // CH):
        sl = pl.ds(s * ZR + k * CH, CH)
        pltpu.sync_copy(acc_sh.at[sl], buf_v)

        @pl.when(c == 0)
        def _():
            pltpu.sync_copy(buf_v, cnt1.at[sl])

        @pl.when(c == 1)
        def _():
            pltpu.sync_copy(buf_v, cnt2.at[sl])


PC = 40


def _pipelined_segsum(table, idx2d, dst2d, rows2, acc_sh, sem_g, sem_s):
    pltpu.async_copy(table.at[idx2d.at[0]], rows2.at[0], sem_g)

    def chunk(i, carry):
        p = lax.rem(i, 2)

        @pl.when(i > 0)
        def _():
            pltpu.make_async_copy(rows2.at[1 - p], acc_sh.at[dst2d.at[i]],
                                  sem_s).wait()

        @pl.when(i + 1 < PC)
        def _():
            pltpu.async_copy(table.at[idx2d.at[i + 1]], rows2.at[1 - p], sem_g)

        pltpu.make_async_copy(table.at[idx2d.at[i]], rows2.at[p], sem_g).wait()
        pltpu.async_copy(rows2.at[p], acc_sh.at[dst2d.at[i]], sem_s, add=True)
        return carry

    lax.fori_loop(0, PC, chunk, 0)
    pltpu.make_async_copy(rows2.at[(PC - 1) % 2], acc_sh.at[dst2d.at[0]],
                          sem_s).wait()


@functools.partial(
    pl.kernel,
    mesh=_mesh,
    out_type=(
        jax.ShapeDtypeStruct((NPAD, 128), jnp.float32),
        jax.ShapeDtypeStruct((NPAD, 128), jnp.float32),
    ),
    scratch_types=[
        pltpu.VMEM((PC, CH), jnp.int32),
        pltpu.VMEM((PC, CH), jnp.int32),
        pltpu.VMEM((2, CH, 128), jnp.float32),
        pltpu.VMEM_SHARED((NPAD, 128), jnp.float32),
        pltpu.SemaphoreType.DMA,
        pltpu.SemaphoreType.DMA,
    ],
)
def _sc_segsum_feat_split(xflat, srclo, srchi, dstp, zerosF, aggL, aggR,
                          idx2d, dst2d, rows2, acc_sh, sem_g, sem_s):
    c = lax.axis_index("c")
    s = lax.axis_index("s")
    nch = ECH // NS
    pltpu.sync_copy(zerosF, rows2.at[0])
    for k in range(ZR // CH):
        pltpu.sync_copy(rows2.at[0], acc_sh.at[pl.ds(s * ZR + k * CH, CH)])

    plsc.subcore_barrier()

    for ph in range(nch // PC):
        off = s * nch + ph * PC

        @pl.when(c == 0)
        def _():
            pltpu.sync_copy(srclo.at[pl.ds(off, PC)], idx2d)

        @pl.when(c == 1)
        def _():
            pltpu.sync_copy(srchi.at[pl.ds(off, PC)], idx2d)

        pltpu.sync_copy(dstp.at[pl.ds(off, PC)], dst2d)
        _pipelined_segsum(xflat, idx2d, dst2d, rows2, acc_sh, sem_g, sem_s)

    plsc.subcore_barrier()
    for k in range(ZR // CH):
        sl = pl.ds(s * ZR + k * CH, CH)
        pltpu.sync_copy(acc_sh.at[sl], rows2.at[0])

        @pl.when(c == 0)
        def _():
            pltpu.sync_copy(rows2.at[0], aggL.at[sl])

        @pl.when(c == 1)
        def _():
            pltpu.sync_copy(rows2.at[0], aggR.at[sl])


@functools.partial(
    pl.kernel,
    mesh=_mesh,
    out_type=(
        jax.ShapeDtypeStruct((NPAD, 128), jnp.float32),
        jax.ShapeDtypeStruct((NPAD, 128), jnp.float32),
    ),
    scratch_types=[
        pltpu.VMEM((PC, CH), jnp.int32),
        pltpu.VMEM((PC, CH), jnp.int32),
        pltpu.VMEM((2, CH, 128), jnp.float32),
        pltpu.VMEM_SHARED((NPAD, 128), jnp.float32),
        pltpu.SemaphoreType.DMA,
        pltpu.SemaphoreType.DMA,
    ],
)
def _sc_segsum_edge_split(table, srcp, dstp, zerosF, p0, p1,
                          idx2d, dst2d, rows2, acc_sh, sem_g, sem_s):
    c = lax.axis_index("c")
    s = lax.axis_index("s")
    nch = ECH // (NC * NS)
    w = c * NS + s
    pltpu.sync_copy(zerosF, rows2.at[0])
    for k in range(ZR // CH):
        pltpu.sync_copy(rows2.at[0], acc_sh.at[pl.ds(s * ZR + k * CH, CH)])
    pltpu.sync_copy(srcp.at[pl.ds(w * nch, nch)], idx2d)
    pltpu.sync_copy(dstp.at[pl.ds(w * nch, nch)], dst2d)
    plsc.subcore_barrier()

    _pipelined_segsum(table, idx2d, dst2d, rows2, acc_sh, sem_g, sem_s)

    plsc.subcore_barrier()
    for k in range(ZR // CH):
        sl = pl.ds(s * ZR + k * CH, CH)
        pltpu.sync_copy(acc_sh.at[sl], rows2.at[0])

        @pl.when(c == 0)
        def _():
            pltpu.sync_copy(rows2.at[0], p0.at[sl])

        @pl.when(c == 1)
        def _():
            pltpu.sync_copy(rows2.at[0], p1.at[sl])


BM = 1000


def _tc_layer1_body(x_ref, aggL_ref, aggR_ref, cnt_ref,
                    w1s_ref, w1nL_ref, w1nR_ref, b1_ref, w2s_ref, w2n_ref,
                    s_ref, t_ref, sums_ref, acc_ref):
    i = pl.program_id(0)
    c1 = jnp.maximum(cnt_ref[:, 0:1], 1.0)
    aL = aggL_ref[...] / c1
    aR = aggR_ref[...] / c1
    h = jnp.dot(x_ref[...], w1s_ref[...], preferred_element_type=jnp.float32)
    h = h + jnp.dot(aL, w1nL_ref[...], preferred_element_type=jnp.float32)
    h = h + jnp.dot(aR, w1nR_ref[...], preferred_element_type=jnp.float32)
    h = h + b1_ref[...]
    h = jnp.maximum(h, 0.0)
    s_ref[...] = jnp.dot(h, w2s_ref[...], preferred_element_type=jnp.float32)
    t_ref[...] = jnp.dot(h, w2n_ref[...], preferred_element_type=jnp.float32)
    bsum = jnp.sum(h)
    bsq = jnp.sum(h * h)

    @pl.when(i == 0)
    def _():
        acc_ref[0] = bsum
        acc_ref[1] = bsq

    @pl.when(i > 0)
    def _():
        acc_ref[0] += bsum
        acc_ref[1] += bsq

    @pl.when(i == pl.num_programs(0) - 1)
    def _():
        sums_ref[0, 0] = acc_ref[0]
        sums_ref[0, 1] = acc_ref[1]


def _tc_layer2_body(s_ref, p0_ref, p1_ref, cnt_ref,
                    sums_ref, w2s_ref, w2n_ref, b2_ref, out_ref):
    tot = float(N * D_H)
    ssum = sums_ref[0, 0]
    ssq = sums_ref[0, 1]
    mu = ssum / tot
    var = ssq / tot - mu * mu
    inv = 1.0 / jnp.sqrt(var + 1e-5)
    cs_self = jnp.sum(w2s_ref[...], axis=0, keepdims=True)
    cs_neigh = jnp.sum(w2n_ref[...], axis=0, keepdims=True)
    seg = p0_ref[...] + p1_ref[...]
    c2 = cnt_ref[:, 0:1]
    denom = jnp.maximum(c2, 1.0)
    aggterm = (seg - (c2 * mu) * cs_neigh) / denom
    out_ref[...] = (s_ref[...] - mu * cs_self + aggterm) * inv + b2_ref[...]


def kernel(x, edge_index, edge_index2, W1_self, W1_neigh, b1,
           W2_self, W2_neigh, b2, num_bits, num_grad_bits):
    f32 = jnp.float32
    xflat = jnp.concatenate([x[:, :128], x[:, 128:]], axis=0)
    pad = EP - E
    pidx = jnp.arange(pad, dtype=jnp.int32)
    psrc = pidx % N
    pdst = N + (pidx % 16)
    src1p = jnp.concatenate([edge_index[0], psrc]).reshape(ECH, CH)
    dst1p = jnp.concatenate([edge_index[1], pdst]).reshape(ECH, CH)
    src2p = jnp.concatenate([edge_index2[0], psrc]).reshape(ECH, CH)
    dst2p = jnp.concatenate([edge_index2[1], pdst]).reshape(ECH, CH)
    src1hi = src1p + N
    zerosF = jnp.zeros((CH, 128), f32)
    onesF = jnp.ones((CH, 128), f32)

    cnt1, cnt2 = _sc_counts(dst1p, dst2p, zerosF, onesF)
    aggL, aggR = _sc_segsum_feat_split(xflat, src1p, src1hi, dst1p, zerosF)

    grid = N // BM
    full = lambda shape: pl.BlockSpec(shape, lambda i: (0, 0))
    row_blk = lambda w: pl.BlockSpec((BM, w), lambda i: (i, 0))
    S, T, sums = pl.pallas_call(
        _tc_layer1_body,
        grid=(grid,),
        in_specs=[
            row_blk(D_IN), row_blk(128), row_blk(128), row_blk(128),
            full((D_IN, D_H)), full((128, D_H)), full((128, D_H)),
            full((1, D_H)), full((D_H, D_OUT)), full((D_H, D_OUT)),
        ],
        out_specs=[
            row_blk(D_OUT), row_blk(D_OUT),
            pl.BlockSpec(memory_space=pltpu.SMEM),
        ],
        out_shape=[
            jax.ShapeDtypeStruct((N, D_OUT), f32),
            jax.ShapeDtypeStruct((N, D_OUT), f32),
            jax.ShapeDtypeStruct((1, 2), f32),
        ],
        scratch_shapes=[pltpu.SMEM((2,), f32)],
    )(x, aggL, aggR, cnt1,
      W1_self, W1_neigh[:128], W1_neigh[128:], b1.reshape(1, D_H),
      W2_self, W2_neigh)

    p0, p1 = _sc_segsum_edge_split(T, src2p, dst2p, zerosF)

    out = pl.pallas_call(
        _tc_layer2_body,
        grid=(grid,),
        in_specs=[
            row_blk(D_OUT), row_blk(D_OUT), row_blk(D_OUT), row_blk(128),
            pl.BlockSpec(memory_space=pltpu.SMEM),
            full((D_H, D_OUT)), full((D_H, D_OUT)), full((1, D_OUT)),
        ],
        out_specs=row_blk(D_OUT),
        out_shape=jax.ShapeDtypeStruct((N, D_OUT), f32),
    )(S, p0, p1, cnt2, sums, W2_self, W2_neigh, b2.reshape(1, D_OUT))

    return out
